# baseline (speedup 1.0000x reference)
import functools

import jax
import jax.numpy as jnp
from jax import lax
from jax.experimental import pallas as pl
from jax.experimental.pallas import tpu as pltpu, tpu_sc as plsc

WIDTH = 1024
TOTAL_ROWS = 4 * 4096

_info = plsc.get_sparse_core_info()
_NC, _NS = _info.num_cores, _info.num_subcores
NUM_WORKERS = _NC * _NS
ROWS_PER_WORKER = TOTAL_ROWS // NUM_WORKERS
GROUP = 16
NUM_GROUPS = ROWS_PER_WORKER // GROUP

_mesh = plsc.VectorSubcoreMesh(core_axis_name="c", subcore_axis_name="s")


@functools.partial(
    pl.kernel,
    mesh=_mesh,
    out_type=jax.ShapeDtypeStruct((TOTAL_ROWS, WIDTH), jnp.float32),
    scratch_types=[
        pltpu.VMEM((2, WIDTH), jnp.float32),
        pltpu.VMEM((NUM_GROUPS, GROUP), jnp.int32),
        pltpu.VMEM((GROUP, WIDTH), jnp.float32),
        pltpu.SemaphoreType.DMA((2,)),
    ],
)
def _lookup_kernel(ids_hbm, table_hbm, out_hbm, table_v, idx_v, drain_v, sem):
    wid = lax.axis_index("s") * _NC + lax.axis_index("c")
    base = wid * ROWS_PER_WORKER

    pltpu.sync_copy(ids_hbm.at[wid], idx_v)
    pltpu.sync_copy(table_hbm, table_v)

    def drain_group(par):
        pltpu.make_async_copy(
            out_hbm.at[pl.ds(base, GROUP)], drain_v, sem.at[par]).wait()

    def group_step(g, _):
        par = lax.rem(g, 2)

        @pl.when(g >= 2)
        def _wait():
            drain_group(par)

        idv = idx_v[g, pl.ds(0, GROUP)]
        for r in range(GROUP):
            rid = idv[r]
            pltpu.async_copy(
                table_v.at[pl.ds(rid, 1)],
                out_hbm.at[pl.ds(base + g * GROUP + r, 1)],
                sem.at[par])
        return _

    lax.fori_loop(0, NUM_GROUPS, group_step, None)

    drain_group(0)
    drain_group(1)


def kernel(token_type_ids, token_type_table):
    ids = token_type_ids.reshape(-1).astype(jnp.int32)
    ids = ids.reshape(NUM_WORKERS, NUM_GROUPS, GROUP)
    return _lookup_kernel(ids, token_type_table)


# speedup vs baseline: 1.0046x; 1.0046x over previous
import functools

import jax
import jax.numpy as jnp
from jax import lax
from jax.experimental import pallas as pl
from jax.experimental.pallas import tpu as pltpu, tpu_sc as plsc

WIDTH = 1024
TOTAL_ROWS = 4 * 4096

_info = plsc.get_sparse_core_info()
_NC, _NS = _info.num_cores, _info.num_subcores
NUM_WORKERS = _NC * _NS
ROWS_PER_WORKER = TOTAL_ROWS // NUM_WORKERS
GROUP = 16
NUM_GROUPS = ROWS_PER_WORKER // GROUP

_mesh = plsc.VectorSubcoreMesh(core_axis_name="c", subcore_axis_name="s")


@functools.partial(
    pl.kernel,
    mesh=_mesh,
    out_type=jax.ShapeDtypeStruct((TOTAL_ROWS, WIDTH), jnp.float32),
    scratch_types=[
        pltpu.VMEM((2, WIDTH), jnp.float32),
        pltpu.VMEM((NUM_GROUPS, GROUP), jnp.int32),
        pltpu.VMEM((GROUP, WIDTH), jnp.float32),
        pltpu.SemaphoreType.DMA((4,)),
    ],
)
def _lookup_kernel(ids_hbm, table_hbm, out_hbm, table_v, idx_v, drain_v, sem):
    wid = lax.axis_index("s") * _NC + lax.axis_index("c")
    base = wid * ROWS_PER_WORKER

    pltpu.sync_copy(ids_hbm.at[wid], idx_v)
    pltpu.sync_copy(table_hbm, table_v)

    def drain_group(par):
        pltpu.make_async_copy(
            out_hbm.at[pl.ds(base, GROUP)], drain_v, sem.at[par]).wait()

    def group_step(g, _):
        par = lax.rem(g, 4)

        @pl.when(g >= 4)
        def _wait():
            drain_group(par)

        idv = idx_v[g, pl.ds(0, GROUP)]
        for r in range(GROUP):
            rid = idv[r]
            pltpu.async_copy(
                table_v.at[pl.ds(rid, 1)],
                out_hbm.at[pl.ds(base + g * GROUP + r, 1)],
                sem.at[par])
        return _

    lax.fori_loop(0, NUM_GROUPS, group_step, None)

    for k in range(4):
        drain_group(k)


def kernel(token_type_ids, token_type_table):
    ids = token_type_ids.reshape(-1).astype(jnp.int32)
    ids = ids.reshape(NUM_WORKERS, NUM_GROUPS, GROUP)
    return _lookup_kernel(ids, token_type_table)
